# Initial kernel scaffold; baseline (speedup 1.0000x reference)
import jax
import jax.numpy as jnp
from jax.experimental import pallas as pl


def kernel(features, params):
    raise NotImplementedError("write your pallas kernel here")


# speedup vs baseline: 8.0863x; 8.0863x over previous
import functools

import jax
import jax.numpy as jnp
from jax import lax
from jax.experimental import pallas as pl

B, N, F = 1024, 128, 16
KNN = 7
EPS = 1e-3


def _fold_bn_matmul(w, bnp):
    g, b, m, v = bnp
    s = g / jnp.sqrt(v + EPS)
    return w * s[None, :], (b - m * s)[None, :]


def _topk_onehot(d):
    iota_j = lax.broadcasted_iota(jnp.int32, (N, N), 1).astype(jnp.float32)
    dwork = d
    ohs = []
    for k in range(KNN + 1):
        mval = jnp.min(dwork, axis=1, keepdims=True)
        idxk = jnp.min(jnp.where(dwork == mval, iota_j, float(N)), axis=1,
                       keepdims=True)
        oh = iota_j == idxk
        if k > 0:
            ohs.append(oh.astype(jnp.float32))
        dwork = jnp.where(oh, jnp.float32(jnp.inf), dwork)
    return jnp.concatenate(ohs, axis=0)


def _edge_conv(pts_d, fts, wd, wb, t1, w2, t2, w3, t3, wsc, tsc):
    G = _topk_onehot(pts_d)
    knn = jnp.dot(G, fts, preferred_element_type=jnp.float32)
    u = jnp.dot(fts, wd, preferred_element_type=jnp.float32)
    ut = jnp.concatenate([u] * KNN, axis=0)
    h = jax.nn.relu(ut + jnp.dot(knn, wb, preferred_element_type=jnp.float32)
                    + t1)
    h = jax.nn.relu(jnp.dot(h, w2, preferred_element_type=jnp.float32) + t2)
    h = jax.nn.relu(jnp.dot(h, w3, preferred_element_type=jnp.float32) + t3)
    c3 = h.shape[-1]
    hm = jnp.mean(h.reshape(KNN, N, c3), axis=0)
    sc = jnp.dot(fts, wsc, preferred_element_type=jnp.float32) + tsc
    return jax.nn.relu(sc + hm)


def _net_kernel(f_ref,
                s0, t0,
                wd0, wb0, t10, w20, t20, w30, t30, wsc0, tsc0,
                wd1, wb1, t11, w21, t21, w31, t31, wsc1, tsc1,
                fcw, fcb, ow, ob,
                o_ref):
    f = f_ref[0]
    fts = f * s0[...] + t0[...]
    eta = f[:, 0:1] * jnp.cos(f[:, 1:2])
    phi = f[:, 0:1] * jnp.sin(f[:, 1:2])
    red = jnp.sum(f, axis=1, keepdims=True)
    mask = (red != 0.0).astype(jnp.float32)
    cshift = 1e9 * (1.0 - mask)

    iota_i = lax.broadcasted_iota(jnp.int32, (N, N), 0)
    iota_j = lax.broadcasted_iota(jnp.int32, (N, N), 1)
    eye = (iota_i == iota_j).astype(jnp.float32)

    def row(col):
        return jnp.sum(eye * col, axis=0, keepdims=True)

    pe = cshift + eta
    pp = cshift + phi
    rA = pe * pe + pp * pp
    mm = pe * row(pe) + pp * row(pp)
    d1 = rA - 2.0 * mm + row(rA)
    fts = _edge_conv(d1, fts, wd0[...], wb0[...], t10[...], w20[...],
                     t20[...], w30[...], t30[...], wsc0[...], tsc0[...])

    pts = cshift + fts
    rA2 = jnp.sum(pts * pts, axis=1, keepdims=True)
    mm2 = lax.dot_general(pts, pts, (((1,), (1,)), ((), ())),
                          preferred_element_type=jnp.float32)
    d2 = rA2 - 2.0 * mm2 + row(rA2)
    fts = _edge_conv(d2, fts, wd1[...], wb1[...], t11[...], w21[...],
                     t21[...], w31[...], t31[...], wsc1[...], tsc1[...])

    pool = jnp.mean(fts * mask, axis=0, keepdims=True)
    h = jax.nn.relu(jnp.dot(pool, fcw[...],
                            preferred_element_type=jnp.float32) + fcb[...])
    lg = jnp.dot(h, ow[...], preferred_element_type=jnp.float32) + ob[...]
    lg = lg - jnp.max(lg, axis=1, keepdims=True)
    e = jnp.exp(lg)
    o_ref[0] = e / jnp.sum(e, axis=1, keepdims=True)


def _prep_weights(params):
    g0, b0, m0, v0 = params["bn0"]
    s0 = (g0 / jnp.sqrt(v0 + EPS))[None, :]
    t0 = (b0 - m0 * (g0 / jnp.sqrt(v0 + EPS)))[None, :]
    ws = [s0, t0]
    for layer in params["layers"]:
        w1, t1 = _fold_bn_matmul(layer["ws"][0], layer["bns"][0])
        c_in = layer["ws"][0].shape[0] // 2
        wt, wb = w1[:c_in], w1[c_in:]
        w2, t2 = _fold_bn_matmul(layer["ws"][1], layer["bns"][1])
        w3, t3 = _fold_bn_matmul(layer["ws"][2], layer["bns"][2])
        wsc, tsc = _fold_bn_matmul(layer["wsc"], layer["bnsc"])
        ws += [wt - wb, wb, t1, w2, t2, w3, t3, wsc, tsc]
    ws += [params["fc_w"], params["fc_b"][None, :], params["out_w"],
           params["out_b"][None, :]]
    return ws


@functools.partial(jax.jit, static_argnames=("interpret",))
def _run(features, params, interpret=False):
    ws = _prep_weights(params)

    def const_spec(a):
        nd = a.ndim
        return pl.BlockSpec(a.shape, lambda i, _nd=nd: (0,) * _nd)

    out = pl.pallas_call(
        _net_kernel,
        grid=(B,),
        in_specs=[pl.BlockSpec((1, N, F), lambda i: (i, 0, 0))]
        + [const_spec(a) for a in ws],
        out_specs=pl.BlockSpec((1, 1, 5), lambda i: (i, 0, 0)),
        out_shape=jax.ShapeDtypeStruct((B, 1, 5), jnp.float32),
        interpret=interpret,
    )(features, *ws)
    return out.reshape(B, 5)


def kernel(features, params):
    return _run(features, params)


# speedup vs baseline: 10.3155x; 1.2757x over previous
import functools

import jax
import jax.numpy as jnp
from jax import lax
from jax.experimental import pallas as pl

B, N, F = 1024, 128, 16
KNN = 7
EPS = 1e-3


def _fold_bn_matmul(w, bnp):
    g, b, m, v = bnp
    s = g / jnp.sqrt(v + EPS)
    return w * s[None, :], (b - m * s)[None, :]


def _topk_onehot(d):
    iota_j = lax.broadcasted_iota(jnp.int32, (N, N), 1).astype(jnp.float32)
    dwork = d
    ohs = []
    for k in range(KNN + 1):
        mval = jnp.min(dwork, axis=1, keepdims=True)
        idxk = jnp.min(jnp.where(dwork == mval, iota_j, float(N)), axis=1,
                       keepdims=True)
        oh = iota_j == idxk
        if k > 0:
            ohs.append(oh.astype(jnp.float32))
        dwork = jnp.where(oh, jnp.float32(jnp.inf), dwork)
    return jnp.concatenate(ohs, axis=0)


def _edge_conv(pts_d, fts, wd, wb, t1, w2, t2, w3, t3, wsc, tsc):
    G = _topk_onehot(pts_d)
    knn = jnp.dot(G, fts, preferred_element_type=jnp.float32)
    u = jnp.dot(fts, wd, preferred_element_type=jnp.float32)
    ut = jnp.concatenate([u] * KNN, axis=0)
    h = jax.nn.relu(ut + jnp.dot(knn, wb, preferred_element_type=jnp.float32)
                    + t1)
    h = jax.nn.relu(jnp.dot(h, w2, preferred_element_type=jnp.float32) + t2)
    h = jax.nn.relu(jnp.dot(h, w3, preferred_element_type=jnp.float32) + t3)
    c3 = h.shape[-1]
    hm = jnp.mean(h.reshape(KNN, N, c3), axis=0)
    sc = jnp.dot(fts, wsc, preferred_element_type=jnp.float32) + tsc
    return jax.nn.relu(sc + hm)


BB = 4


def _net_kernel(f_ref,
                s0, t0,
                wd0, wb0, t10, w20, t20, w30, t30, wsc0, tsc0,
                wd1, wb1, t11, w21, t21, w31, t31, wsc1, tsc1,
                fcw, fcb, ow, ob,
                o_ref):
    for s in range(BB):
        _sample_body(f_ref[s], s0, t0,
                     wd0, wb0, t10, w20, t20, w30, t30, wsc0, tsc0,
                     wd1, wb1, t11, w21, t21, w31, t31, wsc1, tsc1,
                     fcw, fcb, ow, ob, o_ref, s)


def _sample_body(f,
                 s0, t0,
                 wd0, wb0, t10, w20, t20, w30, t30, wsc0, tsc0,
                 wd1, wb1, t11, w21, t21, w31, t31, wsc1, tsc1,
                 fcw, fcb, ow, ob,
                 o_ref, s):
    fts = f * s0[...] + t0[...]
    eta = f[:, 0:1] * jnp.cos(f[:, 1:2])
    phi = f[:, 0:1] * jnp.sin(f[:, 1:2])
    red = jnp.sum(f, axis=1, keepdims=True)
    mask = (red != 0.0).astype(jnp.float32)
    cshift = 1e9 * (1.0 - mask)

    iota_i = lax.broadcasted_iota(jnp.int32, (N, N), 0)
    iota_j = lax.broadcasted_iota(jnp.int32, (N, N), 1)
    eye = (iota_i == iota_j).astype(jnp.float32)

    def row(col):
        return jnp.sum(eye * col, axis=0, keepdims=True)

    pe = cshift + eta
    pp = cshift + phi
    rA = pe * pe + pp * pp
    mm = pe * row(pe) + pp * row(pp)
    d1 = rA - 2.0 * mm + row(rA)
    fts = _edge_conv(d1, fts, wd0[...], wb0[...], t10[...], w20[...],
                     t20[...], w30[...], t30[...], wsc0[...], tsc0[...])

    pts = cshift + fts
    rA2 = jnp.sum(pts * pts, axis=1, keepdims=True)
    mm2 = lax.dot_general(pts, pts, (((1,), (1,)), ((), ())),
                          preferred_element_type=jnp.float32)
    d2 = rA2 - 2.0 * mm2 + row(rA2)
    fts = _edge_conv(d2, fts, wd1[...], wb1[...], t11[...], w21[...],
                     t21[...], w31[...], t31[...], wsc1[...], tsc1[...])

    pool = jnp.mean(fts * mask, axis=0, keepdims=True)
    h = jax.nn.relu(jnp.dot(pool, fcw[...],
                            preferred_element_type=jnp.float32) + fcb[...])
    lg = jnp.dot(h, ow[...], preferred_element_type=jnp.float32) + ob[...]
    lg = lg - jnp.max(lg, axis=1, keepdims=True)
    e = jnp.exp(lg)
    o_ref[s] = e / jnp.sum(e, axis=1, keepdims=True)


def _prep_weights(params):
    g0, b0, m0, v0 = params["bn0"]
    s0 = (g0 / jnp.sqrt(v0 + EPS))[None, :]
    t0 = (b0 - m0 * (g0 / jnp.sqrt(v0 + EPS)))[None, :]
    ws = [s0, t0]
    for layer in params["layers"]:
        w1, t1 = _fold_bn_matmul(layer["ws"][0], layer["bns"][0])
        c_in = layer["ws"][0].shape[0] // 2
        wt, wb = w1[:c_in], w1[c_in:]
        w2, t2 = _fold_bn_matmul(layer["ws"][1], layer["bns"][1])
        w3, t3 = _fold_bn_matmul(layer["ws"][2], layer["bns"][2])
        wsc, tsc = _fold_bn_matmul(layer["wsc"], layer["bnsc"])
        ws += [wt - wb, wb, t1, w2, t2, w3, t3, wsc, tsc]
    ws += [params["fc_w"], params["fc_b"][None, :], params["out_w"],
           params["out_b"][None, :]]
    return ws


@functools.partial(jax.jit, static_argnames=("interpret",))
def _run(features, params, interpret=False):
    ws = _prep_weights(params)

    def const_spec(a):
        nd = a.ndim
        return pl.BlockSpec(a.shape, lambda i, _nd=nd: (0,) * _nd)

    out = pl.pallas_call(
        _net_kernel,
        grid=(B // BB,),
        in_specs=[pl.BlockSpec((BB, N, F), lambda i: (i, 0, 0))]
        + [const_spec(a) for a in ws],
        out_specs=pl.BlockSpec((BB, 1, 5), lambda i: (i, 0, 0)),
        out_shape=jax.ShapeDtypeStruct((B, 1, 5), jnp.float32),
        interpret=interpret,
    )(features, *ws)
    return out.reshape(B, 5)


def kernel(features, params):
    return _run(features, params)


# speedup vs baseline: 20.1428x; 1.9527x over previous
import functools

import jax
import jax.numpy as jnp
from jax import lax
from jax.experimental import pallas as pl

B, N, F = 1024, 128, 16
KNN = 7
EPS = 1e-3
BB = 4


def _fold_bn_matmul(w, bnp):
    g, b, m, v = bnp
    s = g / jnp.sqrt(v + EPS)
    return w * s[None, :], (b - m * s)[None, :]


def _topk_onehot(d):
    iota_j = lax.broadcasted_iota(jnp.int32, (BB, N, N), 2).astype(jnp.float32)
    dwork = d
    ohs = []
    for k in range(KNN + 1):
        mval = jnp.min(dwork, axis=2, keepdims=True)
        idxk = jnp.min(jnp.where(dwork == mval, iota_j, float(N)), axis=2,
                       keepdims=True)
        oh = iota_j == idxk
        if k > 0:
            ohs.append(oh.astype(jnp.float32))
        dwork = jnp.where(oh, jnp.float32(jnp.inf), dwork)
    return jnp.concatenate(ohs, axis=1)


def _edge_conv(d, fts, wd, wb, t1, w2, t2, w3, t3, wsc, tsc):
    G = _topk_onehot(d)
    knn = lax.dot_general(G, fts, (((2,), (1,)), ((0,), (0,))),
                          preferred_element_type=jnp.float32)
    c_in = fts.shape[-1]
    fts2 = fts.reshape(BB * N, c_in)
    u = jnp.dot(fts2, wd, preferred_element_type=jnp.float32)
    c1 = u.shape[-1]
    ut = jnp.concatenate([u.reshape(BB, 1, N, c1)] * KNN, axis=1)
    ut = ut.reshape(BB * KNN * N, c1)
    kf = knn.reshape(BB * KNN * N, c_in)
    h = jax.nn.relu(ut + jnp.dot(kf, wb, preferred_element_type=jnp.float32)
                    + t1)
    h = jax.nn.relu(jnp.dot(h, w2, preferred_element_type=jnp.float32) + t2)
    h = jax.nn.relu(jnp.dot(h, w3, preferred_element_type=jnp.float32) + t3)
    c3 = h.shape[-1]
    hm = jnp.mean(h.reshape(BB, KNN, N, c3), axis=1)
    sc = jnp.dot(fts2, wsc, preferred_element_type=jnp.float32) + tsc
    return jax.nn.relu(sc.reshape(BB, N, c3) + hm)


def _net_kernel(f_ref,
                s0, t0,
                wd0, wb0, t10, w20, t20, w30, t30, wsc0, tsc0,
                wd1, wb1, t11, w21, t21, w31, t31, wsc1, tsc1,
                fcw, fcb, ow, ob,
                o_ref):
    f = f_ref[...]
    fts = f * s0[...] + t0[...]
    eta = f[:, :, 0:1] * jnp.cos(f[:, :, 1:2])
    phi = f[:, :, 0:1] * jnp.sin(f[:, :, 1:2])
    onesF = jnp.full((F, 1), 1.0, jnp.float32)
    red = lax.dot_general(f.reshape(BB * N, F), onesF,
                          (((1,), (0,)), ((), ())),
                          preferred_element_type=jnp.float32)
    mask = (red != 0.0).astype(jnp.float32).reshape(BB, N, 1)
    cshift = 1e9 * (1.0 - mask)

    iota_i = lax.broadcasted_iota(jnp.int32, (N, N), 0)
    iota_j = lax.broadcasted_iota(jnp.int32, (N, N), 1)
    eye = (iota_i == iota_j).astype(jnp.float32)[None]

    def row(col):
        return jnp.sum(eye * col, axis=1, keepdims=True)

    pe = cshift + eta
    pp = cshift + phi
    rA = pe * pe + pp * pp
    mm = pe * row(pe) + pp * row(pp)
    d1 = rA - 2.0 * mm + row(rA)
    fts = _edge_conv(d1, fts, wd0[...], wb0[...], t10[...], w20[...],
                     t20[...], w30[...], t30[...], wsc0[...], tsc0[...])

    pts = cshift + fts
    rA2 = jnp.sum(pts * pts, axis=2, keepdims=True)
    mm2 = lax.dot_general(pts, pts, (((2,), (2,)), ((0,), (0,))),
                          preferred_element_type=jnp.float32)
    d2 = rA2 - 2.0 * mm2 + row(rA2)
    fts = _edge_conv(d2, fts, wd1[...], wb1[...], t11[...], w21[...],
                     t21[...], w31[...], t31[...], wsc1[...], tsc1[...])

    pool = jnp.mean(fts * mask, axis=1)
    h = jax.nn.relu(jnp.dot(pool, fcw[...],
                            preferred_element_type=jnp.float32) + fcb[...])
    lg = jnp.dot(h, ow[...], preferred_element_type=jnp.float32) + ob[...]
    lg = lg - jnp.max(lg, axis=1, keepdims=True)
    e = jnp.exp(lg)
    p = e / jnp.sum(e, axis=1, keepdims=True)
    o_ref[...] = p.reshape(BB, 1, 5)


def _prep_weights(params):
    g0, b0, m0, v0 = params["bn0"]
    s0 = (g0 / jnp.sqrt(v0 + EPS))[None, :]
    t0 = (b0 - m0 * (g0 / jnp.sqrt(v0 + EPS)))[None, :]
    ws = [s0, t0]
    for layer in params["layers"]:
        w1, t1 = _fold_bn_matmul(layer["ws"][0], layer["bns"][0])
        c_in = layer["ws"][0].shape[0] // 2
        wt, wb = w1[:c_in], w1[c_in:]
        w2, t2 = _fold_bn_matmul(layer["ws"][1], layer["bns"][1])
        w3, t3 = _fold_bn_matmul(layer["ws"][2], layer["bns"][2])
        wsc, tsc = _fold_bn_matmul(layer["wsc"], layer["bnsc"])
        ws += [wt - wb, wb, t1, w2, t2, w3, t3, wsc, tsc]
    ws += [params["fc_w"], params["fc_b"][None, :], params["out_w"],
           params["out_b"][None, :]]
    return ws


@functools.partial(jax.jit, static_argnames=("interpret",))
def _run(features, params, interpret=False):
    ws = _prep_weights(params)

    def const_spec(a):
        nd = a.ndim
        return pl.BlockSpec(a.shape, lambda i, _nd=nd: (0,) * _nd)

    out = pl.pallas_call(
        _net_kernel,
        grid=(B // BB,),
        in_specs=[pl.BlockSpec((BB, N, F), lambda i: (i, 0, 0))]
        + [const_spec(a) for a in ws],
        out_specs=pl.BlockSpec((BB, 1, 5), lambda i: (i, 0, 0)),
        out_shape=jax.ShapeDtypeStruct((B, 1, 5), jnp.float32),
        interpret=interpret,
    )(features, *ws)
    return out.reshape(B, 5)


def kernel(features, params):
    return _run(features, params)


# speedup vs baseline: 25.8726x; 1.2845x over previous
import functools

import jax
import jax.numpy as jnp
from jax import lax
from jax.experimental import pallas as pl

B, N, F = 1024, 128, 16
KNN = 7
EPS = 1e-3
BB = 4


def _fold_bn_matmul(w, bnp):
    g, b, m, v = bnp
    s = g / jnp.sqrt(v + EPS)
    return w * s[None, :], (b - m * s)[None, :]


def _topk_onehot(d):
    iota_i = lax.broadcasted_iota(jnp.int32, (BB, N, N), 1).astype(jnp.float32)
    dwork = d
    ohs = []
    for k in range(KNN + 1):
        mval = jnp.min(dwork, axis=1, keepdims=True)
        idxk = jnp.min(jnp.where(dwork == mval, iota_i, float(N)), axis=1,
                       keepdims=True)
        oh = iota_i == idxk
        if k > 0:
            ohs.append(oh.astype(jnp.float32))
        dwork = jnp.where(oh, jnp.float32(jnp.inf), dwork)
    return jnp.concatenate(ohs, axis=2)


def _edge_conv(d, fts, wd, wb, t1, w2, t2, w3, t3, wsc, tsc):
    G = _topk_onehot(d)
    knn = lax.dot_general(G, fts, (((1,), (1,)), ((0,), (0,))),
                          preferred_element_type=jnp.float32)
    c_in = fts.shape[-1]
    fts2 = fts.reshape(BB * N, c_in)
    u = jnp.dot(fts2, wd, preferred_element_type=jnp.float32)
    c1 = u.shape[-1]
    ut = jnp.concatenate([u.reshape(BB, 1, N, c1)] * KNN, axis=1)
    ut = ut.reshape(BB * KNN * N, c1)
    kf = knn.reshape(BB * KNN * N, c_in)
    h = jax.nn.relu(ut + jnp.dot(kf, wb, preferred_element_type=jnp.float32)
                    + t1)
    h = jax.nn.relu(jnp.dot(h, w2, preferred_element_type=jnp.float32) + t2)
    h = jax.nn.relu(jnp.dot(h, w3, preferred_element_type=jnp.float32) + t3)
    c3 = h.shape[-1]
    hm = jnp.mean(h.reshape(BB, KNN, N, c3), axis=1)
    sc = jnp.dot(fts2, wsc, preferred_element_type=jnp.float32) + tsc
    return jax.nn.relu(sc.reshape(BB, N, c3) + hm)


def _net_kernel(f_ref,
                s0, t0,
                wd0, wb0, t10, w20, t20, w30, t30, wsc0, tsc0,
                wd1, wb1, t11, w21, t21, w31, t31, wsc1, tsc1,
                fcw, fcb, ow, ob,
                o_ref):
    f = f_ref[...]
    fts = f * s0[...] + t0[...]
    eta = f[:, :, 0:1] * jnp.cos(f[:, :, 1:2])
    phi = f[:, :, 0:1] * jnp.sin(f[:, :, 1:2])
    onesF = jnp.full((F, 1), 1.0, jnp.float32)
    red = lax.dot_general(f.reshape(BB * N, F), onesF,
                          (((1,), (0,)), ((), ())),
                          preferred_element_type=jnp.float32)
    mask = (red != 0.0).astype(jnp.float32).reshape(BB, N, 1)
    cshift = 1e9 * (1.0 - mask)

    pe = cshift + eta
    pp = cshift + phi
    p01 = jnp.concatenate([pe, pp], axis=2)
    rA = pe * pe + pp * pp
    mm = lax.dot_general(p01, p01, (((2,), (2,)), ((0,), (0,))),
                         preferred_element_type=jnp.float32)
    d1 = rA - 2.0 * mm
    fts = _edge_conv(d1, fts, wd0[...], wb0[...], t10[...], w20[...],
                     t20[...], w30[...], t30[...], wsc0[...], tsc0[...])

    pts = cshift + fts
    rA2 = jnp.sum(pts * pts, axis=2, keepdims=True)
    mm2 = lax.dot_general(pts, pts, (((2,), (2,)), ((0,), (0,))),
                          preferred_element_type=jnp.float32)
    d2 = rA2 - 2.0 * mm2
    fts = _edge_conv(d2, fts, wd1[...], wb1[...], t11[...], w21[...],
                     t21[...], w31[...], t31[...], wsc1[...], tsc1[...])

    pool = jnp.mean(fts * mask, axis=1)
    h = jax.nn.relu(jnp.dot(pool, fcw[...],
                            preferred_element_type=jnp.float32) + fcb[...])
    lg = jnp.dot(h, ow[...], preferred_element_type=jnp.float32) + ob[...]
    lg = lg - jnp.max(lg, axis=1, keepdims=True)
    e = jnp.exp(lg)
    p = e / jnp.sum(e, axis=1, keepdims=True)
    o_ref[...] = p.reshape(BB, 1, 5)


def _prep_weights(params):
    g0, b0, m0, v0 = params["bn0"]
    s0 = (g0 / jnp.sqrt(v0 + EPS))[None, :]
    t0 = (b0 - m0 * (g0 / jnp.sqrt(v0 + EPS)))[None, :]
    ws = [s0, t0]
    for layer in params["layers"]:
        w1, t1 = _fold_bn_matmul(layer["ws"][0], layer["bns"][0])
        c_in = layer["ws"][0].shape[0] // 2
        wt, wb = w1[:c_in], w1[c_in:]
        w2, t2 = _fold_bn_matmul(layer["ws"][1], layer["bns"][1])
        w3, t3 = _fold_bn_matmul(layer["ws"][2], layer["bns"][2])
        wsc, tsc = _fold_bn_matmul(layer["wsc"], layer["bnsc"])
        ws += [wt - wb, wb, t1, w2, t2, w3, t3, wsc, tsc]
    ws += [params["fc_w"], params["fc_b"][None, :], params["out_w"],
           params["out_b"][None, :]]
    return ws


@functools.partial(jax.jit, static_argnames=("interpret",))
def _run(features, params, interpret=False):
    ws = _prep_weights(params)

    def const_spec(a):
        nd = a.ndim
        return pl.BlockSpec(a.shape, lambda i, _nd=nd: (0,) * _nd)

    out = pl.pallas_call(
        _net_kernel,
        grid=(B // BB,),
        in_specs=[pl.BlockSpec((BB, N, F), lambda i: (i, 0, 0))]
        + [const_spec(a) for a in ws],
        out_specs=pl.BlockSpec((BB, 1, 5), lambda i: (i, 0, 0)),
        out_shape=jax.ShapeDtypeStruct((B, 1, 5), jnp.float32),
        interpret=interpret,
    )(features, *ws)
    return out.reshape(B, 5)


def kernel(features, params):
    return _run(features, params)


# speedup vs baseline: 37.3881x; 1.4451x over previous
import functools

import jax
import jax.numpy as jnp
from jax import lax
from jax.experimental import pallas as pl

B, N, F = 1024, 128, 16
KNN = 7
EPS = 1e-3
BB = 4


def _fold_bn_matmul(w, bnp):
    g, b, m, v = bnp
    s = g / jnp.sqrt(v + EPS)
    return w * s[None, :], (b - m * s)[:, None]


def _bcast(w):
    return jnp.broadcast_to(w[None], (BB,) + w.shape)


def _topk_onehot(d, iota_if):
    dwork = d
    ohs = []
    for k in range(KNN + 1):
        mval = jnp.min(dwork, axis=1, keepdims=True)
        idxk = jnp.min(jnp.where(dwork == mval, iota_if, float(N)), axis=1,
                       keepdims=True)
        oh = iota_if == idxk
        if k > 0:
            ohs.append(oh.astype(jnp.float32))
        dwork = jnp.where(oh, jnp.float32(jnp.inf), dwork)
    return jnp.concatenate(ohs, axis=2)


def _edge_conv(d, iota_if, ftsT, wdT, wbT, t1, w2T, t2, w3T, t3, wscT, tsc):
    G = _topk_onehot(d, iota_if)
    knnT = lax.dot_general(ftsT, G, (((2,), (1,)), ((0,), (0,))),
                           preferred_element_type=jnp.float32)
    uT = lax.dot_general(_bcast(wdT), ftsT, (((2,), (1,)), ((0,), (0,))),
                         preferred_element_type=jnp.float32)
    utT = jnp.concatenate([uT] * KNN, axis=2)
    h = jax.nn.relu(
        utT
        + lax.dot_general(_bcast(wbT), knnT, (((2,), (1,)), ((0,), (0,))),
                          preferred_element_type=jnp.float32) + t1)
    h = jax.nn.relu(
        lax.dot_general(_bcast(w2T), h, (((2,), (1,)), ((0,), (0,))),
                        preferred_element_type=jnp.float32) + t2)
    h = jax.nn.relu(
        lax.dot_general(_bcast(w3T), h, (((2,), (1,)), ((0,), (0,))),
                        preferred_element_type=jnp.float32) + t3)
    hm = h[:, :, 0:N]
    for k in range(1, KNN):
        hm = hm + h[:, :, k * N:(k + 1) * N]
    hm = hm * jnp.float32(1.0 / KNN)
    scT = lax.dot_general(_bcast(wscT), ftsT, (((2,), (1,)), ((0,), (0,))),
                          preferred_element_type=jnp.float32) + tsc
    return jax.nn.relu(scT + hm)


def _net_kernel(fT_ref,
                s0, t0,
                wd0, wb0, t10, w20, t20, w30, t30, wsc0, tsc0,
                wd1, wb1, t11, w21, t21, w31, t31, wsc1, tsc1,
                fcw, fcb, ow, ob,
                o_ref):
    fT = fT_ref[...]
    ftsT = fT * s0[...] + t0[...]
    etaR = fT[:, 0:1, :] * jnp.cos(fT[:, 1:2, :])
    phiR = fT[:, 0:1, :] * jnp.sin(fT[:, 1:2, :])
    redR = jnp.sum(fT, axis=1, keepdims=True)
    maskR = (redR != 0.0).astype(jnp.float32)
    cshiftR = 1e9 * (1.0 - maskR)

    iota_i = lax.broadcasted_iota(jnp.int32, (BB, N, N), 1)
    iota_if = iota_i.astype(jnp.float32)
    eye3 = (iota_i == lax.broadcasted_iota(jnp.int32, (BB, N, N), 2)
            ).astype(jnp.float32)

    def col(rowvec):
        return lax.dot_general(eye3, rowvec, (((2,), (2,)), ((0,), (0,))),
                               preferred_element_type=jnp.float32)

    peR = cshiftR + etaR
    ppR = cshiftR + phiR
    p01T = jnp.concatenate([peR, ppR], axis=1)
    rAR = peR * peR + ppR * ppR
    mm = lax.dot_general(p01T, p01T, (((1,), (1,)), ((0,), (0,))),
                         preferred_element_type=jnp.float32)
    d1 = col(rAR) - 2.0 * mm
    ftsT = _edge_conv(d1, iota_if, ftsT, wd0[...], wb0[...], t10[...],
                      w20[...], t20[...], w30[...], t30[...], wsc0[...],
                      tsc0[...])

    ptsT = cshiftR + ftsT
    rA2R = jnp.sum(ptsT * ptsT, axis=1, keepdims=True)
    mm2 = lax.dot_general(ptsT, ptsT, (((1,), (1,)), ((0,), (0,))),
                          preferred_element_type=jnp.float32)
    d2 = col(rA2R) - 2.0 * mm2
    ftsT = _edge_conv(d2, iota_if, ftsT, wd1[...], wb1[...], t11[...],
                      w21[...], t21[...], w31[...], t31[...], wsc1[...],
                      tsc1[...])

    ftsmT = ftsT * maskR
    poolc = lax.dot_general(ftsmT, _bcast(jnp.full((N, 1), 1.0 / N,
                                                   jnp.float32)),
                            (((2,), (1,)), ((0,), (0,))),
                            preferred_element_type=jnp.float32)
    h = jax.nn.relu(
        lax.dot_general(_bcast(fcw), poolc, (((2,), (1,)), ((0,), (0,))),
                        preferred_element_type=jnp.float32) + fcb[...])
    lg = lax.dot_general(_bcast(ow), h, (((2,), (1,)), ((0,), (0,))),
                         preferred_element_type=jnp.float32) + ob[...]
    lg = lg - jnp.max(lg, axis=1, keepdims=True)
    e = jnp.exp(lg)
    o_ref[...] = e / jnp.sum(e, axis=1, keepdims=True)


def _prep_weights(params):
    g0, b0, m0, v0 = params["bn0"]
    s0 = (g0 / jnp.sqrt(v0 + EPS))[:, None]
    t0 = (b0 - m0 * g0 / jnp.sqrt(v0 + EPS))[:, None]
    ws = [s0, t0]
    for layer in params["layers"]:
        w1, t1 = _fold_bn_matmul(layer["ws"][0], layer["bns"][0])
        c_in = layer["ws"][0].shape[0] // 2
        wt, wb = w1[:c_in], w1[c_in:]
        w2, t2 = _fold_bn_matmul(layer["ws"][1], layer["bns"][1])
        w3, t3 = _fold_bn_matmul(layer["ws"][2], layer["bns"][2])
        wsc, tsc = _fold_bn_matmul(layer["wsc"], layer["bnsc"])
        ws += [(wt - wb).T, wb.T, t1, w2.T, t2, w3.T, t3, wsc.T, tsc]
    ws += [params["fc_w"].T, params["fc_b"][:, None], params["out_w"].T,
           params["out_b"][:, None]]
    return ws


@functools.partial(jax.jit, static_argnames=("interpret",))
def _run(features, params, interpret=False):
    ws = _prep_weights(params)
    fT = features.transpose(0, 2, 1)

    def const_spec(a):
        nd = a.ndim
        return pl.BlockSpec(a.shape, lambda i, _nd=nd: (0,) * _nd)

    out = pl.pallas_call(
        _net_kernel,
        grid=(B // BB,),
        in_specs=[pl.BlockSpec((BB, F, N), lambda i: (i, 0, 0))]
        + [const_spec(a) for a in ws],
        out_specs=pl.BlockSpec((BB, 5, 1), lambda i: (i, 0, 0)),
        out_shape=jax.ShapeDtypeStruct((B, 5, 1), jnp.float32),
        interpret=interpret,
    )(fT, *ws)
    return out.reshape(B, 5)


def kernel(features, params):
    return _run(features, params)


# speedup vs baseline: 43.0421x; 1.1512x over previous
import functools

import jax
import jax.numpy as jnp
from jax import lax
from jax.experimental import pallas as pl

B, N, F = 1024, 128, 16
KNN = 7
EPS = 1e-3
BB = 4


def _fold_bn_matmul(w, bnp):
    g, b, m, v = bnp
    s = g / jnp.sqrt(v + EPS)
    return w * s[None, :], (b - m * s)[:, None]


def _bcast(w):
    return jnp.broadcast_to(w[None], (BB,) + w.shape)


def _topk_onehot(d, iota_if):
    b = lax.bitcast_convert_type(d, jnp.int32)
    s = jnp.where(b >= 0, b, b ^ jnp.int32(0x7FFFFFFF))
    km = (s & jnp.int32(-128)) | iota_if
    ohs = []
    for k in range(KNN + 1):
        mval = jnp.min(km, axis=1, keepdims=True)
        oh = km == mval
        if k > 0:
            ohs.append(oh.astype(jnp.float32))
        km = jnp.where(oh, jnp.int32(0x7FFFFFFF), km)
    return jnp.concatenate(ohs, axis=2)


def _edge_conv(d, iota_if, ftsT, wdT, wbT, t1, w2T, t2, w3T, t3, wscT, tsc):
    G = _topk_onehot(d, iota_if)
    knnT = lax.dot_general(ftsT, G, (((2,), (1,)), ((0,), (0,))),
                           preferred_element_type=jnp.float32)
    uT = lax.dot_general(_bcast(wdT), ftsT, (((2,), (1,)), ((0,), (0,))),
                         preferred_element_type=jnp.float32) + t1
    utT = jnp.concatenate([uT] * KNN, axis=2)
    h = jax.nn.relu(
        utT
        + lax.dot_general(_bcast(wbT), knnT, (((2,), (1,)), ((0,), (0,))),
                          preferred_element_type=jnp.float32))
    h = jax.nn.relu(
        lax.dot_general(_bcast(w2T), h, (((2,), (1,)), ((0,), (0,))),
                        preferred_element_type=jnp.float32) + t2)
    h = jax.nn.relu(
        lax.dot_general(_bcast(w3T), h, (((2,), (1,)), ((0,), (0,))),
                        preferred_element_type=jnp.float32) + t3)
    hm = h[:, :, 0:N]
    for k in range(1, KNN):
        hm = hm + h[:, :, k * N:(k + 1) * N]
    hm = hm * jnp.float32(1.0 / KNN)
    scT = lax.dot_general(_bcast(wscT), ftsT, (((2,), (1,)), ((0,), (0,))),
                          preferred_element_type=jnp.float32) + tsc
    return jax.nn.relu(scT + hm)


def _net_kernel(fT_ref,
                s0, t0,
                wd0, wb0, t10, w20, t20, w30, t30, wsc0, tsc0,
                wd1, wb1, t11, w21, t21, w31, t31, wsc1, tsc1,
                fcw, fcb, ow, ob,
                o_ref):
    fT = fT_ref[...]
    ftsT = fT * s0[...] + t0[...]
    etaR = fT[:, 0:1, :] * jnp.cos(fT[:, 1:2, :])
    phiR = fT[:, 0:1, :] * jnp.sin(fT[:, 1:2, :])
    redR = jnp.sum(fT, axis=1, keepdims=True)
    maskR = (redR != 0.0).astype(jnp.float32)
    cshiftR = 1e9 * (1.0 - maskR)

    iota_i = lax.broadcasted_iota(jnp.int32, (BB, N, N), 1)
    iota_if = iota_i
    eye3 = (iota_i == lax.broadcasted_iota(jnp.int32, (BB, N, N), 2)
            ).astype(jnp.float32)

    def col(rowvec):
        return lax.dot_general(eye3, rowvec, (((2,), (2,)), ((0,), (0,))),
                               preferred_element_type=jnp.float32)

    peR = cshiftR + etaR
    ppR = cshiftR + phiR
    p01T = jnp.concatenate([peR, ppR], axis=1)
    rAR = peR * peR + ppR * ppR
    mm = lax.dot_general(p01T, p01T, (((1,), (1,)), ((0,), (0,))),
                         preferred_element_type=jnp.float32)
    d1 = col(rAR) - 2.0 * mm
    ftsT = _edge_conv(d1, iota_if, ftsT, wd0[...], wb0[...], t10[...],
                      w20[...], t20[...], w30[...], t30[...], wsc0[...],
                      tsc0[...])

    ptsT = cshiftR + ftsT
    rA2R = jnp.sum(ptsT * ptsT, axis=1, keepdims=True)
    mm2 = lax.dot_general(ptsT, ptsT, (((1,), (1,)), ((0,), (0,))),
                          preferred_element_type=jnp.float32)
    d2 = col(rA2R) - 2.0 * mm2
    ftsT = _edge_conv(d2, iota_if, ftsT, wd1[...], wb1[...], t11[...],
                      w21[...], t21[...], w31[...], t31[...], wsc1[...],
                      tsc1[...])

    ftsmT = ftsT * maskR
    poolc = lax.dot_general(ftsmT, _bcast(jnp.full((N, 1), 1.0 / N,
                                                   jnp.float32)),
                            (((2,), (1,)), ((0,), (0,))),
                            preferred_element_type=jnp.float32)
    h = jax.nn.relu(
        lax.dot_general(_bcast(fcw), poolc, (((2,), (1,)), ((0,), (0,))),
                        preferred_element_type=jnp.float32) + fcb[...])
    lg = lax.dot_general(_bcast(ow), h, (((2,), (1,)), ((0,), (0,))),
                         preferred_element_type=jnp.float32) + ob[...]
    lg = lg - jnp.max(lg, axis=1, keepdims=True)
    e = jnp.exp(lg)
    o_ref[...] = e / jnp.sum(e, axis=1, keepdims=True)


def _prep_weights(params):
    g0, b0, m0, v0 = params["bn0"]
    s0 = (g0 / jnp.sqrt(v0 + EPS))[:, None]
    t0 = (b0 - m0 * g0 / jnp.sqrt(v0 + EPS))[:, None]
    ws = [s0, t0]
    for layer in params["layers"]:
        w1, t1 = _fold_bn_matmul(layer["ws"][0], layer["bns"][0])
        c_in = layer["ws"][0].shape[0] // 2
        wt, wb = w1[:c_in], w1[c_in:]
        w2, t2 = _fold_bn_matmul(layer["ws"][1], layer["bns"][1])
        w3, t3 = _fold_bn_matmul(layer["ws"][2], layer["bns"][2])
        wsc, tsc = _fold_bn_matmul(layer["wsc"], layer["bnsc"])
        ws += [(wt - wb).T, wb.T, t1, w2.T, t2, w3.T, t3, wsc.T, tsc]
    ws += [params["fc_w"].T, params["fc_b"][:, None], params["out_w"].T,
           params["out_b"][:, None]]
    return ws


@functools.partial(jax.jit, static_argnames=("interpret",))
def _run(features, params, interpret=False):
    ws = _prep_weights(params)
    fT = features.transpose(0, 2, 1)

    def const_spec(a):
        nd = a.ndim
        return pl.BlockSpec(a.shape, lambda i, _nd=nd: (0,) * _nd)

    out = pl.pallas_call(
        _net_kernel,
        grid=(B // BB,),
        in_specs=[pl.BlockSpec((BB, F, N), lambda i: (i, 0, 0))]
        + [const_spec(a) for a in ws],
        out_specs=pl.BlockSpec((BB, 5, 1), lambda i: (i, 0, 0)),
        out_shape=jax.ShapeDtypeStruct((B, 5, 1), jnp.float32),
        interpret=interpret,
    )(fT, *ws)
    return out.reshape(B, 5)


def kernel(features, params):
    return _run(features, params)


# speedup vs baseline: 58.7318x; 1.3645x over previous
import functools

import jax
import jax.numpy as jnp
from jax import lax
from jax.experimental import pallas as pl

B, N, F = 1024, 128, 16
KNN = 7
EPS = 1e-3
BB = 8


def _fold_bn_matmul(w, bnp):
    g, b, m, v = bnp
    s = g / jnp.sqrt(v + EPS)
    return w * s[None, :], (b - m * s)[:, None]


def _bcast(w):
    return jnp.broadcast_to(w[None], (BB,) + w.shape)


def _topk_onehot(d, iota_if):
    b = lax.bitcast_convert_type(d, jnp.int32)
    s = jnp.where(b >= 0, b, b ^ jnp.int32(0x7FFFFFFF))
    km = (s & jnp.int32(-128)) | iota_if
    ohs = []
    for k in range(KNN + 1):
        mval = jnp.min(km, axis=1, keepdims=True)
        oh = km == mval
        if k > 0:
            ohs.append(oh.astype(jnp.float32))
        km = jnp.where(oh, jnp.int32(0x7FFFFFFF), km)
    return jnp.concatenate(ohs, axis=2)


def _edge_conv(d, iota_if, ftsT, wdT, wbT, t1, w2T, t2, w3T, t3, wscT, tsc):
    G = _topk_onehot(d, iota_if)
    knnT = lax.dot_general(ftsT, G, (((2,), (1,)), ((0,), (0,))),
                           preferred_element_type=jnp.float32)
    uT = lax.dot_general(_bcast(wdT), ftsT, (((2,), (1,)), ((0,), (0,))),
                         preferred_element_type=jnp.float32) + t1
    utT = jnp.concatenate([uT] * KNN, axis=2)
    h = jax.nn.relu(
        utT
        + lax.dot_general(_bcast(wbT), knnT, (((2,), (1,)), ((0,), (0,))),
                          preferred_element_type=jnp.float32))
    h = jax.nn.relu(
        lax.dot_general(_bcast(w2T), h, (((2,), (1,)), ((0,), (0,))),
                        preferred_element_type=jnp.float32) + t2)
    h = jax.nn.relu(
        lax.dot_general(_bcast(w3T), h, (((2,), (1,)), ((0,), (0,))),
                        preferred_element_type=jnp.float32) + t3)
    hm = h[:, :, 0:N]
    for k in range(1, KNN):
        hm = hm + h[:, :, k * N:(k + 1) * N]
    hm = hm * jnp.float32(1.0 / KNN)
    scT = lax.dot_general(_bcast(wscT), ftsT, (((2,), (1,)), ((0,), (0,))),
                          preferred_element_type=jnp.float32) + tsc
    return jax.nn.relu(scT + hm)


def _net_kernel(fT_ref,
                s0, t0,
                wd0, wb0, t10, w20, t20, w30, t30, wsc0, tsc0,
                wd1, wb1, t11, w21, t21, w31, t31, wsc1, tsc1,
                fcw, fcb, ow, ob,
                o_ref):
    fT = fT_ref[...]
    ftsT = fT * s0[...] + t0[...]
    etaR = fT[:, 0:1, :] * jnp.cos(fT[:, 1:2, :])
    phiR = fT[:, 0:1, :] * jnp.sin(fT[:, 1:2, :])
    redR = jnp.sum(fT, axis=1, keepdims=True)
    maskR = (redR != 0.0).astype(jnp.float32)
    cshiftR = 1e9 * (1.0 - maskR)

    iota_i = lax.broadcasted_iota(jnp.int32, (BB, N, N), 1)
    iota_if = iota_i
    eye3 = (iota_i == lax.broadcasted_iota(jnp.int32, (BB, N, N), 2)
            ).astype(jnp.float32)

    def col(rowvec):
        return lax.dot_general(eye3, rowvec, (((2,), (2,)), ((0,), (0,))),
                               preferred_element_type=jnp.float32)

    peR = cshiftR + etaR
    ppR = cshiftR + phiR
    p01T = jnp.concatenate([peR, ppR], axis=1)
    rAR = peR * peR + ppR * ppR
    mm = lax.dot_general(p01T, p01T, (((1,), (1,)), ((0,), (0,))),
                         preferred_element_type=jnp.float32)
    d1 = col(rAR) - 2.0 * mm
    ftsT = _edge_conv(d1, iota_if, ftsT, wd0[...], wb0[...], t10[...],
                      w20[...], t20[...], w30[...], t30[...], wsc0[...],
                      tsc0[...])

    ptsT = cshiftR + ftsT
    rA2R = jnp.sum(ptsT * ptsT, axis=1, keepdims=True)
    mm2 = lax.dot_general(ptsT, ptsT, (((1,), (1,)), ((0,), (0,))),
                          preferred_element_type=jnp.float32)
    d2 = col(rA2R) - 2.0 * mm2
    ftsT = _edge_conv(d2, iota_if, ftsT, wd1[...], wb1[...], t11[...],
                      w21[...], t21[...], w31[...], t31[...], wsc1[...],
                      tsc1[...])

    ftsmT = ftsT * maskR
    poolc = lax.dot_general(ftsmT, _bcast(jnp.full((N, 1), 1.0 / N,
                                                   jnp.float32)),
                            (((2,), (1,)), ((0,), (0,))),
                            preferred_element_type=jnp.float32)
    h = jax.nn.relu(
        lax.dot_general(_bcast(fcw), poolc, (((2,), (1,)), ((0,), (0,))),
                        preferred_element_type=jnp.float32) + fcb[...])
    lg = lax.dot_general(_bcast(ow), h, (((2,), (1,)), ((0,), (0,))),
                         preferred_element_type=jnp.float32) + ob[...]
    lg = lg - jnp.max(lg, axis=1, keepdims=True)
    e = jnp.exp(lg)
    o_ref[...] = e / jnp.sum(e, axis=1, keepdims=True)


def _prep_weights(params):
    g0, b0, m0, v0 = params["bn0"]
    s0 = (g0 / jnp.sqrt(v0 + EPS))[:, None]
    t0 = (b0 - m0 * g0 / jnp.sqrt(v0 + EPS))[:, None]
    ws = [s0, t0]
    for layer in params["layers"]:
        w1, t1 = _fold_bn_matmul(layer["ws"][0], layer["bns"][0])
        c_in = layer["ws"][0].shape[0] // 2
        wt, wb = w1[:c_in], w1[c_in:]
        w2, t2 = _fold_bn_matmul(layer["ws"][1], layer["bns"][1])
        w3, t3 = _fold_bn_matmul(layer["ws"][2], layer["bns"][2])
        wsc, tsc = _fold_bn_matmul(layer["wsc"], layer["bnsc"])
        ws += [(wt - wb).T, wb.T, t1, w2.T, t2, w3.T, t3, wsc.T, tsc]
    ws += [params["fc_w"].T, params["fc_b"][:, None], params["out_w"].T,
           params["out_b"][:, None]]
    return ws


@functools.partial(jax.jit, static_argnames=("interpret",))
def _run(features, params, interpret=False):
    ws = _prep_weights(params)
    fT = features.transpose(0, 2, 1)

    def const_spec(a):
        nd = a.ndim
        return pl.BlockSpec(a.shape, lambda i, _nd=nd: (0,) * _nd)

    out = pl.pallas_call(
        _net_kernel,
        grid=(B // BB,),
        in_specs=[pl.BlockSpec((BB, F, N), lambda i: (i, 0, 0))]
        + [const_spec(a) for a in ws],
        out_specs=pl.BlockSpec((BB, 5, 1), lambda i: (i, 0, 0)),
        out_shape=jax.ShapeDtypeStruct((B, 5, 1), jnp.float32),
        interpret=interpret,
    )(fT, *ws)
    return out.reshape(B, 5)


def kernel(features, params):
    return _run(features, params)


# speedup vs baseline: 67.4039x; 1.1477x over previous
import functools

import jax
import jax.numpy as jnp
from jax import lax
from jax.experimental import pallas as pl

B, N, F = 1024, 128, 16
KNN = 7
EPS = 1e-3
BB = 16


def _fold_bn_matmul(w, bnp):
    g, b, m, v = bnp
    s = g / jnp.sqrt(v + EPS)
    return w * s[None, :], (b - m * s)[:, None]


def _bcast(w):
    return jnp.broadcast_to(w[None], (BB,) + w.shape)


def _topk_onehot(d, iota_if):
    b = lax.bitcast_convert_type(d, jnp.int32)
    s = jnp.where(b >= 0, b, b ^ jnp.int32(0x7FFFFFFF))
    km = (s & jnp.int32(-128)) | iota_if
    ohs = []
    for k in range(KNN + 1):
        mval = jnp.min(km, axis=1, keepdims=True)
        oh = km == mval
        if k > 0:
            ohs.append(oh.astype(jnp.float32))
        km = jnp.where(oh, jnp.int32(0x7FFFFFFF), km)
    return jnp.concatenate(ohs, axis=2)


def _edge_conv(d, iota_if, ftsT, wdT, wbT, t1, w2T, t2, w3T, t3, wscT, tsc):
    G = _topk_onehot(d, iota_if)
    knnT = lax.dot_general(ftsT, G, (((2,), (1,)), ((0,), (0,))),
                           preferred_element_type=jnp.float32)
    uT = lax.dot_general(_bcast(wdT), ftsT, (((2,), (1,)), ((0,), (0,))),
                         preferred_element_type=jnp.float32) + t1
    utT = jnp.concatenate([uT] * KNN, axis=2)
    h = jax.nn.relu(
        utT
        + lax.dot_general(_bcast(wbT), knnT, (((2,), (1,)), ((0,), (0,))),
                          preferred_element_type=jnp.float32))
    h = jax.nn.relu(
        lax.dot_general(_bcast(w2T), h, (((2,), (1,)), ((0,), (0,))),
                        preferred_element_type=jnp.float32) + t2)
    h = jax.nn.relu(
        lax.dot_general(_bcast(w3T), h, (((2,), (1,)), ((0,), (0,))),
                        preferred_element_type=jnp.float32) + t3)
    hm = h[:, :, 0:N]
    for k in range(1, KNN):
        hm = hm + h[:, :, k * N:(k + 1) * N]
    hm = hm * jnp.float32(1.0 / KNN)
    scT = lax.dot_general(_bcast(wscT), ftsT, (((2,), (1,)), ((0,), (0,))),
                          preferred_element_type=jnp.float32) + tsc
    return jax.nn.relu(scT + hm)


def _net_kernel(fT_ref,
                s0, t0,
                wd0, wb0, t10, w20, t20, w30, t30, wsc0, tsc0,
                wd1, wb1, t11, w21, t21, w31, t31, wsc1, tsc1,
                fcw, fcb, ow, ob,
                o_ref):
    fT = fT_ref[...]
    ftsT = fT * s0[...] + t0[...]
    etaR = fT[:, 0:1, :] * jnp.cos(fT[:, 1:2, :])
    phiR = fT[:, 0:1, :] * jnp.sin(fT[:, 1:2, :])
    redR = jnp.sum(fT, axis=1, keepdims=True)
    maskR = (redR != 0.0).astype(jnp.float32)
    cshiftR = 1e9 * (1.0 - maskR)

    iota_i = lax.broadcasted_iota(jnp.int32, (BB, N, N), 1)
    iota_if = iota_i
    eye3 = (iota_i == lax.broadcasted_iota(jnp.int32, (BB, N, N), 2)
            ).astype(jnp.float32)

    def col(rowvec):
        return lax.dot_general(eye3, rowvec, (((2,), (2,)), ((0,), (0,))),
                               preferred_element_type=jnp.float32)

    peR = cshiftR + etaR
    ppR = cshiftR + phiR
    p01T = jnp.concatenate([peR, ppR], axis=1)
    rAR = peR * peR + ppR * ppR
    mm = lax.dot_general(p01T, p01T, (((1,), (1,)), ((0,), (0,))),
                         preferred_element_type=jnp.float32)
    d1 = col(rAR) - 2.0 * mm
    ftsT = _edge_conv(d1, iota_if, ftsT, wd0[...], wb0[...], t10[...],
                      w20[...], t20[...], w30[...], t30[...], wsc0[...],
                      tsc0[...])

    ptsT = cshiftR + ftsT
    rA2R = jnp.sum(ptsT * ptsT, axis=1, keepdims=True)
    mm2 = lax.dot_general(ptsT, ptsT, (((1,), (1,)), ((0,), (0,))),
                          preferred_element_type=jnp.float32)
    d2 = col(rA2R) - 2.0 * mm2
    ftsT = _edge_conv(d2, iota_if, ftsT, wd1[...], wb1[...], t11[...],
                      w21[...], t21[...], w31[...], t31[...], wsc1[...],
                      tsc1[...])

    ftsmT = ftsT * maskR
    poolc = lax.dot_general(ftsmT, _bcast(jnp.full((N, 1), 1.0 / N,
                                                   jnp.float32)),
                            (((2,), (1,)), ((0,), (0,))),
                            preferred_element_type=jnp.float32)
    h = jax.nn.relu(
        lax.dot_general(_bcast(fcw), poolc, (((2,), (1,)), ((0,), (0,))),
                        preferred_element_type=jnp.float32) + fcb[...])
    lg = lax.dot_general(_bcast(ow), h, (((2,), (1,)), ((0,), (0,))),
                         preferred_element_type=jnp.float32) + ob[...]
    lg = lg - jnp.max(lg, axis=1, keepdims=True)
    e = jnp.exp(lg)
    o_ref[...] = e / jnp.sum(e, axis=1, keepdims=True)


def _prep_weights(params):
    g0, b0, m0, v0 = params["bn0"]
    s0 = (g0 / jnp.sqrt(v0 + EPS))[:, None]
    t0 = (b0 - m0 * g0 / jnp.sqrt(v0 + EPS))[:, None]
    ws = [s0, t0]
    for layer in params["layers"]:
        w1, t1 = _fold_bn_matmul(layer["ws"][0], layer["bns"][0])
        c_in = layer["ws"][0].shape[0] // 2
        wt, wb = w1[:c_in], w1[c_in:]
        w2, t2 = _fold_bn_matmul(layer["ws"][1], layer["bns"][1])
        w3, t3 = _fold_bn_matmul(layer["ws"][2], layer["bns"][2])
        wsc, tsc = _fold_bn_matmul(layer["wsc"], layer["bnsc"])
        ws += [(wt - wb).T, wb.T, t1, w2.T, t2, w3.T, t3, wsc.T, tsc]
    ws += [params["fc_w"].T, params["fc_b"][:, None], params["out_w"].T,
           params["out_b"][:, None]]
    return ws


@functools.partial(jax.jit, static_argnames=("interpret",))
def _run(features, params, interpret=False):
    ws = _prep_weights(params)
    fT = features.transpose(0, 2, 1)

    def const_spec(a):
        nd = a.ndim
        return pl.BlockSpec(a.shape, lambda i, _nd=nd: (0,) * _nd)

    out = pl.pallas_call(
        _net_kernel,
        grid=(B // BB,),
        in_specs=[pl.BlockSpec((BB, F, N), lambda i: (i, 0, 0))]
        + [const_spec(a) for a in ws],
        out_specs=pl.BlockSpec((BB, 5, 1), lambda i: (i, 0, 0)),
        out_shape=jax.ShapeDtypeStruct((B, 5, 1), jnp.float32),
        interpret=interpret,
    )(fT, *ws)
    return out.reshape(B, 5)


def kernel(features, params):
    return _run(features, params)


# speedup vs baseline: 72.1899x; 1.0710x over previous
import functools

import jax
import jax.numpy as jnp
from jax import lax
from jax.experimental import pallas as pl

B, N, F = 1024, 128, 16
KNN = 7
EPS = 1e-3
BB = 32


def _fold_bn_matmul(w, bnp):
    g, b, m, v = bnp
    s = g / jnp.sqrt(v + EPS)
    return w * s[None, :], (b - m * s)[:, None]


def _bcast(w):
    return jnp.broadcast_to(w[None], (BB,) + w.shape)


def _topk_onehot(d, iota_if):
    b = lax.bitcast_convert_type(d, jnp.int32)
    s = jnp.where(b >= 0, b, b ^ jnp.int32(0x7FFFFFFF))
    km = (s & jnp.int32(-128)) | iota_if
    ohs = []
    for k in range(KNN + 1):
        mval = jnp.min(km, axis=1, keepdims=True)
        oh = km == mval
        if k > 0:
            ohs.append(oh.astype(jnp.float32))
        km = jnp.where(oh, jnp.int32(0x7FFFFFFF), km)
    return jnp.concatenate(ohs, axis=2)


def _edge_conv(d, iota_if, ftsT, wdT, wbT, t1, w2T, t2, w3T, t3, wscT, tsc):
    G = _topk_onehot(d, iota_if)
    knnT = lax.dot_general(ftsT, G, (((2,), (1,)), ((0,), (0,))),
                           preferred_element_type=jnp.float32)
    uT = lax.dot_general(_bcast(wdT), ftsT, (((2,), (1,)), ((0,), (0,))),
                         preferred_element_type=jnp.float32) + t1
    utT = jnp.concatenate([uT] * KNN, axis=2)
    h = jax.nn.relu(
        utT
        + lax.dot_general(_bcast(wbT), knnT, (((2,), (1,)), ((0,), (0,))),
                          preferred_element_type=jnp.float32))
    h = jax.nn.relu(
        lax.dot_general(_bcast(w2T), h, (((2,), (1,)), ((0,), (0,))),
                        preferred_element_type=jnp.float32) + t2)
    h = jax.nn.relu(
        lax.dot_general(_bcast(w3T), h, (((2,), (1,)), ((0,), (0,))),
                        preferred_element_type=jnp.float32) + t3)
    hm = h[:, :, 0:N]
    for k in range(1, KNN):
        hm = hm + h[:, :, k * N:(k + 1) * N]
    hm = hm * jnp.float32(1.0 / KNN)
    scT = lax.dot_general(_bcast(wscT), ftsT, (((2,), (1,)), ((0,), (0,))),
                          preferred_element_type=jnp.float32) + tsc
    return jax.nn.relu(scT + hm)


def _net_kernel(fT_ref,
                s0, t0,
                wd0, wb0, t10, w20, t20, w30, t30, wsc0, tsc0,
                wd1, wb1, t11, w21, t21, w31, t31, wsc1, tsc1,
                fcw, fcb, ow, ob,
                o_ref):
    fT = fT_ref[...]
    ftsT = fT * s0[...] + t0[...]
    etaR = fT[:, 0:1, :] * jnp.cos(fT[:, 1:2, :])
    phiR = fT[:, 0:1, :] * jnp.sin(fT[:, 1:2, :])
    redR = jnp.sum(fT, axis=1, keepdims=True)
    maskR = (redR != 0.0).astype(jnp.float32)
    cshiftR = 1e9 * (1.0 - maskR)

    iota_i = lax.broadcasted_iota(jnp.int32, (BB, N, N), 1)
    iota_if = iota_i
    eye3 = (iota_i == lax.broadcasted_iota(jnp.int32, (BB, N, N), 2)
            ).astype(jnp.float32)

    def col(rowvec):
        return lax.dot_general(eye3, rowvec, (((2,), (2,)), ((0,), (0,))),
                               preferred_element_type=jnp.float32)

    peR = cshiftR + etaR
    ppR = cshiftR + phiR
    p01T = jnp.concatenate([peR, ppR], axis=1)
    rAR = peR * peR + ppR * ppR
    mm = lax.dot_general(p01T, p01T, (((1,), (1,)), ((0,), (0,))),
                         preferred_element_type=jnp.float32)
    d1 = col(rAR) - 2.0 * mm
    ftsT = _edge_conv(d1, iota_if, ftsT, wd0[...], wb0[...], t10[...],
                      w20[...], t20[...], w30[...], t30[...], wsc0[...],
                      tsc0[...])

    ptsT = cshiftR + ftsT
    rA2R = jnp.sum(ptsT * ptsT, axis=1, keepdims=True)
    mm2 = lax.dot_general(ptsT, ptsT, (((1,), (1,)), ((0,), (0,))),
                          preferred_element_type=jnp.float32)
    d2 = col(rA2R) - 2.0 * mm2
    ftsT = _edge_conv(d2, iota_if, ftsT, wd1[...], wb1[...], t11[...],
                      w21[...], t21[...], w31[...], t31[...], wsc1[...],
                      tsc1[...])

    ftsmT = ftsT * maskR
    poolc = lax.dot_general(ftsmT, _bcast(jnp.full((N, 1), 1.0 / N,
                                                   jnp.float32)),
                            (((2,), (1,)), ((0,), (0,))),
                            preferred_element_type=jnp.float32)
    h = jax.nn.relu(
        lax.dot_general(_bcast(fcw), poolc, (((2,), (1,)), ((0,), (0,))),
                        preferred_element_type=jnp.float32) + fcb[...])
    lg = lax.dot_general(_bcast(ow), h, (((2,), (1,)), ((0,), (0,))),
                         preferred_element_type=jnp.float32) + ob[...]
    lg = lg - jnp.max(lg, axis=1, keepdims=True)
    e = jnp.exp(lg)
    o_ref[...] = e / jnp.sum(e, axis=1, keepdims=True)


def _prep_weights(params):
    g0, b0, m0, v0 = params["bn0"]
    s0 = (g0 / jnp.sqrt(v0 + EPS))[:, None]
    t0 = (b0 - m0 * g0 / jnp.sqrt(v0 + EPS))[:, None]
    ws = [s0, t0]
    for layer in params["layers"]:
        w1, t1 = _fold_bn_matmul(layer["ws"][0], layer["bns"][0])
        c_in = layer["ws"][0].shape[0] // 2
        wt, wb = w1[:c_in], w1[c_in:]
        w2, t2 = _fold_bn_matmul(layer["ws"][1], layer["bns"][1])
        w3, t3 = _fold_bn_matmul(layer["ws"][2], layer["bns"][2])
        wsc, tsc = _fold_bn_matmul(layer["wsc"], layer["bnsc"])
        ws += [(wt - wb).T, wb.T, t1, w2.T, t2, w3.T, t3, wsc.T, tsc]
    ws += [params["fc_w"].T, params["fc_b"][:, None], params["out_w"].T,
           params["out_b"][:, None]]
    return ws


@functools.partial(jax.jit, static_argnames=("interpret",))
def _run(features, params, interpret=False):
    ws = _prep_weights(params)
    fT = features.transpose(0, 2, 1)

    def const_spec(a):
        nd = a.ndim
        return pl.BlockSpec(a.shape, lambda i, _nd=nd: (0,) * _nd)

    out = pl.pallas_call(
        _net_kernel,
        grid=(B // BB,),
        in_specs=[pl.BlockSpec((BB, F, N), lambda i: (i, 0, 0))]
        + [const_spec(a) for a in ws],
        out_specs=pl.BlockSpec((BB, 5, 1), lambda i: (i, 0, 0)),
        out_shape=jax.ShapeDtypeStruct((B, 5, 1), jnp.float32),
        interpret=interpret,
    )(fT, *ws)
    return out.reshape(B, 5)


def kernel(features, params):
    return _run(features, params)


# speedup vs baseline: 74.0741x; 1.0261x over previous
import functools

import jax
import jax.numpy as jnp
from jax import lax
from jax.experimental import pallas as pl

B, N, F = 1024, 128, 16
KNN = 7
EPS = 1e-3
BB = 64


def _fold_bn_matmul(w, bnp):
    g, b, m, v = bnp
    s = g / jnp.sqrt(v + EPS)
    return w * s[None, :], (b - m * s)[:, None]


def _bcast(w):
    return jnp.broadcast_to(w[None], (BB,) + w.shape)


def _topk_onehot(d, iota_if):
    b = lax.bitcast_convert_type(d, jnp.int32)
    s = jnp.where(b >= 0, b, b ^ jnp.int32(0x7FFFFFFF))
    km = (s & jnp.int32(-128)) | iota_if
    ohs = []
    for k in range(KNN + 1):
        mval = jnp.min(km, axis=1, keepdims=True)
        oh = km == mval
        if k > 0:
            ohs.append(oh.astype(jnp.float32))
        km = jnp.where(oh, jnp.int32(0x7FFFFFFF), km)
    return jnp.concatenate(ohs, axis=2)


def _edge_conv(d, iota_if, ftsT, wdT, wbT, t1, w2T, t2, w3T, t3, wscT, tsc):
    G = _topk_onehot(d, iota_if)
    knnT = lax.dot_general(ftsT, G, (((2,), (1,)), ((0,), (0,))),
                           preferred_element_type=jnp.float32)
    uT = lax.dot_general(_bcast(wdT), ftsT, (((2,), (1,)), ((0,), (0,))),
                         preferred_element_type=jnp.float32) + t1
    utT = jnp.concatenate([uT] * KNN, axis=2)
    h = jax.nn.relu(
        utT
        + lax.dot_general(_bcast(wbT), knnT, (((2,), (1,)), ((0,), (0,))),
                          preferred_element_type=jnp.float32))
    h = jax.nn.relu(
        lax.dot_general(_bcast(w2T), h, (((2,), (1,)), ((0,), (0,))),
                        preferred_element_type=jnp.float32) + t2)
    h = jax.nn.relu(
        lax.dot_general(_bcast(w3T), h, (((2,), (1,)), ((0,), (0,))),
                        preferred_element_type=jnp.float32) + t3)
    hm = h[:, :, 0:N]
    for k in range(1, KNN):
        hm = hm + h[:, :, k * N:(k + 1) * N]
    hm = hm * jnp.float32(1.0 / KNN)
    scT = lax.dot_general(_bcast(wscT), ftsT, (((2,), (1,)), ((0,), (0,))),
                          preferred_element_type=jnp.float32) + tsc
    return jax.nn.relu(scT + hm)


def _net_kernel(fT_ref,
                s0, t0,
                wd0, wb0, t10, w20, t20, w30, t30, wsc0, tsc0,
                wd1, wb1, t11, w21, t21, w31, t31, wsc1, tsc1,
                fcw, fcb, ow, ob,
                o_ref):
    fT = fT_ref[...]
    ftsT = fT * s0[...] + t0[...]
    etaR = fT[:, 0:1, :] * jnp.cos(fT[:, 1:2, :])
    phiR = fT[:, 0:1, :] * jnp.sin(fT[:, 1:2, :])
    redR = jnp.sum(fT, axis=1, keepdims=True)
    maskR = (redR != 0.0).astype(jnp.float32)
    cshiftR = 1e9 * (1.0 - maskR)

    iota_i = lax.broadcasted_iota(jnp.int32, (BB, N, N), 1)
    iota_if = iota_i
    eye3 = (iota_i == lax.broadcasted_iota(jnp.int32, (BB, N, N), 2)
            ).astype(jnp.float32)

    def col(rowvec):
        return lax.dot_general(eye3, rowvec, (((2,), (2,)), ((0,), (0,))),
                               preferred_element_type=jnp.float32)

    peR = cshiftR + etaR
    ppR = cshiftR + phiR
    p01T = jnp.concatenate([peR, ppR], axis=1)
    rAR = peR * peR + ppR * ppR
    mm = lax.dot_general(p01T, p01T, (((1,), (1,)), ((0,), (0,))),
                         preferred_element_type=jnp.float32)
    d1 = col(rAR) - 2.0 * mm
    ftsT = _edge_conv(d1, iota_if, ftsT, wd0[...], wb0[...], t10[...],
                      w20[...], t20[...], w30[...], t30[...], wsc0[...],
                      tsc0[...])

    ptsT = cshiftR + ftsT
    rA2R = jnp.sum(ptsT * ptsT, axis=1, keepdims=True)
    mm2 = lax.dot_general(ptsT, ptsT, (((1,), (1,)), ((0,), (0,))),
                          preferred_element_type=jnp.float32)
    d2 = col(rA2R) - 2.0 * mm2
    ftsT = _edge_conv(d2, iota_if, ftsT, wd1[...], wb1[...], t11[...],
                      w21[...], t21[...], w31[...], t31[...], wsc1[...],
                      tsc1[...])

    ftsmT = ftsT * maskR
    poolc = lax.dot_general(ftsmT, _bcast(jnp.full((N, 1), 1.0 / N,
                                                   jnp.float32)),
                            (((2,), (1,)), ((0,), (0,))),
                            preferred_element_type=jnp.float32)
    h = jax.nn.relu(
        lax.dot_general(_bcast(fcw), poolc, (((2,), (1,)), ((0,), (0,))),
                        preferred_element_type=jnp.float32) + fcb[...])
    lg = lax.dot_general(_bcast(ow), h, (((2,), (1,)), ((0,), (0,))),
                         preferred_element_type=jnp.float32) + ob[...]
    lg = lg - jnp.max(lg, axis=1, keepdims=True)
    e = jnp.exp(lg)
    o_ref[...] = e / jnp.sum(e, axis=1, keepdims=True)


def _prep_weights(params):
    g0, b0, m0, v0 = params["bn0"]
    s0 = (g0 / jnp.sqrt(v0 + EPS))[:, None]
    t0 = (b0 - m0 * g0 / jnp.sqrt(v0 + EPS))[:, None]
    ws = [s0, t0]
    for layer in params["layers"]:
        w1, t1 = _fold_bn_matmul(layer["ws"][0], layer["bns"][0])
        c_in = layer["ws"][0].shape[0] // 2
        wt, wb = w1[:c_in], w1[c_in:]
        w2, t2 = _fold_bn_matmul(layer["ws"][1], layer["bns"][1])
        w3, t3 = _fold_bn_matmul(layer["ws"][2], layer["bns"][2])
        wsc, tsc = _fold_bn_matmul(layer["wsc"], layer["bnsc"])
        ws += [(wt - wb).T, wb.T, t1, w2.T, t2, w3.T, t3, wsc.T, tsc]
    ws += [params["fc_w"].T, params["fc_b"][:, None], params["out_w"].T,
           params["out_b"][:, None]]
    return ws


@functools.partial(jax.jit, static_argnames=("interpret",))
def _run(features, params, interpret=False):
    ws = _prep_weights(params)
    fT = features.transpose(0, 2, 1)

    def const_spec(a):
        nd = a.ndim
        return pl.BlockSpec(a.shape, lambda i, _nd=nd: (0,) * _nd)

    out = pl.pallas_call(
        _net_kernel,
        grid=(B // BB,),
        in_specs=[pl.BlockSpec((BB, F, N), lambda i: (i, 0, 0))]
        + [const_spec(a) for a in ws],
        out_specs=pl.BlockSpec((BB, 5, 1), lambda i: (i, 0, 0)),
        out_shape=jax.ShapeDtypeStruct((B, 5, 1), jnp.float32),
        interpret=interpret,
    )(fT, *ws)
    return out.reshape(B, 5)


def kernel(features, params):
    return _run(features, params)
